# baseline (speedup 1.0000x reference)
import functools

import jax
import jax.numpy as jnp
from jax.experimental import pallas as pl
from jax.experimental.pallas import tpu as pltpu

ALPHA = 1.702
LIMIT = 7.0
NC = 4


def _fp4_scaled_bf16(c, s32):
    sign = (c & 8) << 28
    m = c & 7
    e = m >> 1
    m0 = m & 1
    norm = ((s32 + e - 1) << 23) | (m0 << 22)
    sub = jnp.where(m0 == 1, (s32 - 1) << 23, jnp.zeros_like(s32))
    bits = sign | jnp.where(e == 0, sub, norm)
    return jax.lax.bitcast_convert_type(bits, jnp.float32).astype(jnp.bfloat16)


_DN = (((1,), (1,)), ((), ()))


def _moe_body(TM, T, H, I,
              e_ref, base_ref, nrows_ref, tok_ref, wgt_ref,
              hid_ref, q_ref, s_ref, dq_ref, ds_ref, bgu_ref, bd_ref,
              out_ref, xs_ref, ys_ref, wlo_ref, whi_ref, wd_ref):
    t = pl.program_id(0)
    c = pl.program_id(1)
    Hh = H // 2
    CI = 2 * I // NC
    SB = 128

    @pl.when((t == 0) & (c == 0))
    def _():
        out_ref[...] = jnp.zeros_like(out_ref)

    nrows = nrows_ref[t]
    base = base_ref[t]

    @pl.when((nrows > 0) & (c == 0))
    def _():
        def gather(i, _):
            xs_ref[pl.ds(i, 1), :] = hid_ref[pl.ds(tok_ref[base + i], 1), :]
            return 0
        jax.lax.fori_loop(0, nrows, gather, 0)

    @pl.when(nrows > 0)
    def _():
        def dq_gu(i, _):
            r = i * SB
            qi = q_ref[pl.ds(r, SB), :].astype(jnp.int32)
            si = jnp.repeat(s_ref[pl.ds(r, SB), :].astype(jnp.int32), 16, axis=1)
            wlo_ref[pl.ds(r, SB), :] = _fp4_scaled_bf16(qi & 15, si)
            whi_ref[pl.ds(r, SB), :] = _fp4_scaled_bf16(qi >> 4, si)
            return 0
        jax.lax.fori_loop(0, CI // SB, dq_gu, 0)

        dlane = jax.lax.broadcasted_iota(jnp.int32, (SB, CI), 1)
        use_hi = ((dlane >> 1) & 1) == 1

        def dq_d(i, _):
            r = i * SB
            di = jnp.repeat(dq_ref[pl.ds(r, SB), :].astype(jnp.int32), 4, axis=1)
            si = jnp.repeat(ds_ref[pl.ds(r, SB), :].astype(jnp.int32), 64, axis=1)
            nib = jnp.where(use_hi, di >> 4, di & 15)
            wd_ref[pl.ds(r, SB), :] = _fp4_scaled_bf16(nib, si)
            return 0
        jax.lax.fori_loop(0, H // SB, dq_d, 0)

        x = xs_ref[...].astype(jnp.bfloat16)
        x_lo = x[:, :Hh]
        x_hi = x[:, Hh:]

        gu = jax.lax.dot_general(x_lo, wlo_ref[...], _DN, preferred_element_type=jnp.float32)
        gu += jax.lax.dot_general(x_hi, whi_ref[...], _DN, preferred_element_type=jnp.float32)
        gu += bgu_ref[...]

        gate = jnp.minimum(gu, LIMIT)
        up = jnp.clip(gu, -LIMIT, LIMIT)
        glu = gate * jax.nn.sigmoid(gate * ALPHA)
        up1 = jnp.roll(up, -1, axis=1)
        gated = (up1 + 1.0) * glu
        lane = jax.lax.broadcasted_iota(jnp.int32, (TM, CI), 1)
        gated = jnp.where((lane & 1) == 0, gated, 0.0).astype(jnp.bfloat16)

        part = jax.lax.dot_general(gated, wd_ref[...], _DN, preferred_element_type=jnp.float32)

        @pl.when(c == 0)
        def _():
            ys_ref[...] = part

        @pl.when(c > 0)
        def _():
            ys_ref[...] += part

    @pl.when((nrows > 0) & (c == NC - 1))
    def _():
        ys_ref[...] += bd_ref[...]

        def scatter(i, _):
            tok = tok_ref[base + i]
            row = ys_ref[pl.ds(i, 1), :] * wgt_ref[base + i]
            out_ref[pl.ds(tok, 1), :] += row
            return 0
        jax.lax.fori_loop(0, nrows, scatter, 0)


def kernel(hidden_states, router_indices, routing_weights,
           gate_up_qweight, gate_up_scales, down_qweight, down_scales,
           gate_up_proj_bias, down_proj_bias):
    T, H = hidden_states.shape
    E = gate_up_qweight.shape[0]
    I = down_qweight.shape[2] * 2
    TOPK = router_indices.shape[1]
    P = T * TOPK
    TM = 128
    NT = P // TM + E
    Hh = H // 2
    I2 = 2 * I
    CI = I2 // NC

    flat = router_indices.reshape(-1).astype(jnp.int32)
    order = jnp.argsort(flat).astype(jnp.int32)
    tok = (order // TOPK).astype(jnp.int32)
    wgt = routing_weights.reshape(-1)[order]
    counts = jnp.zeros((E,), jnp.int32).at[flat].add(1)
    offsets = jnp.cumsum(counts) - counts
    ntiles = (counts + TM - 1) // TM
    tcum = jnp.cumsum(ntiles)
    first_tile = tcum - ntiles
    tr = jnp.arange(NT, dtype=jnp.int32)
    e_of_t = jnp.clip(jnp.searchsorted(tcum, tr, side="right"), 0, E - 1).astype(jnp.int32)
    local = tr - first_tile[e_of_t]
    base = jnp.clip(offsets[e_of_t] + local * TM, 0, P - 1).astype(jnp.int32)
    nrows = jnp.clip(counts[e_of_t] - local * TM, 0, TM).astype(jnp.int32)

    hidden_de = jnp.swapaxes(hidden_states.reshape(T, Hh, 2), 1, 2).reshape(T, H)
    ds_t = down_scales.reshape(E, H, NC, I // (32 * NC)).transpose(0, 2, 1, 3)
    bgu = gate_up_proj_bias.reshape(E, 1, I2)
    bd = down_proj_bias.reshape(E, 1, H)

    grid_spec = pltpu.PrefetchScalarGridSpec(
        num_scalar_prefetch=5,
        grid=(NT, NC),
        in_specs=[
            pl.BlockSpec((T, H), lambda t, c, *_: (0, 0)),
            pl.BlockSpec((None, CI, Hh), lambda t, c, e_r, *_: (e_r[t], c, 0)),
            pl.BlockSpec((None, CI, Hh // 16), lambda t, c, e_r, *_: (e_r[t], c, 0)),
            pl.BlockSpec((None, H, CI // 4), lambda t, c, e_r, *_: (e_r[t], 0, c)),
            pl.BlockSpec((None, None, H, CI // 64), lambda t, c, e_r, *_: (e_r[t], c, 0, 0)),
            pl.BlockSpec((None, 1, CI), lambda t, c, e_r, *_: (e_r[t], 0, c)),
            pl.BlockSpec((None, 1, H), lambda t, c, e_r, *_: (e_r[t], 0, 0)),
        ],
        out_specs=pl.BlockSpec((T, H), lambda t, c, *_: (0, 0)),
        scratch_shapes=[
            pltpu.VMEM((TM, H), jnp.float32),
            pltpu.VMEM((TM, H), jnp.float32),
            pltpu.VMEM((CI, Hh), jnp.bfloat16),
            pltpu.VMEM((CI, Hh), jnp.bfloat16),
            pltpu.VMEM((H, CI), jnp.bfloat16),
        ],
    )

    body = functools.partial(_moe_body, TM, T, H, I)
    out = pl.pallas_call(
        body,
        grid_spec=grid_spec,
        out_shape=jax.ShapeDtypeStruct((T, H), jnp.float32),
        compiler_params=pltpu.CompilerParams(
            dimension_semantics=("arbitrary", "arbitrary"),
        ),
    )(e_of_t, base, nrows, tok, wgt,
      hidden_de, gate_up_qweight, gate_up_scales, down_qweight, ds_t,
      bgu, bd)
    return out


# speedup vs baseline: 12.0073x; 12.0073x over previous
import functools

import jax
import jax.numpy as jnp
from jax.experimental import pallas as pl
from jax.experimental.pallas import tpu as pltpu

ALPHA = 1.702
LIMIT = 7.0
NC = 4


def _fp4_bf16(c):
    sign = (c & 8) << 28
    m = c & 7
    e = m >> 1
    m0 = m & 1
    norm = ((126 + e) << 23) | (m0 << 22)
    sub = jnp.where(m0 == 1, jnp.full_like(c, 126 << 23), jnp.zeros_like(c))
    bits = sign | jnp.where(e == 0, sub, norm)
    return jax.lax.bitcast_convert_type(bits, jnp.float32).astype(jnp.bfloat16)


_DN = (((1,), (1,)), ((), ()))
_DS = (((1,), (0,)), ((), ()))


def _mmf(a, b):
    return jax.lax.dot_general(a, b, _DS, preferred_element_type=jnp.float32)


def _moe_body(TM, T, H, I,
              e_ref, base_ref, nrows_ref, tok_ref, wgt_ref,
              hid_ref, q_ref, s_ref, dq_ref, ds_ref, bgu_ref, bd_ref,
              r16_ref, r16b_ref, clo_ref, chi_ref,
              out_ref, xs_ref, ys_ref, wlo_ref, whi_ref, wdlo_ref, wdhi_ref):
    t = pl.program_id(0)
    c = pl.program_id(1)
    Hh = H // 2
    CI = 2 * I // NC
    CB = CI // 4
    SB = 128

    @pl.when((t == 0) & (c == 0))
    def _():
        out_ref[...] = jnp.zeros_like(out_ref)

    nrows = nrows_ref[t]
    base = base_ref[t]

    @pl.when((nrows > 0) & (c == 0))
    def _():
        def gather(i, _):
            xs_ref[pl.ds(i, 1), :] = hid_ref[pl.ds(tok_ref[base + i], 1), :]
            return 0
        jax.lax.fori_loop(0, nrows, gather, 0)

    @pl.when(nrows > 0)
    def _():
        def dq_gu(i, _):
            r = i * SB
            qi = q_ref[pl.ds(r, SB), :].astype(jnp.int32)
            sf = jnp.exp2(s_ref[pl.ds(r, SB), :].astype(jnp.float32) - 127.0)
            se = _mmf(sf.astype(jnp.bfloat16), r16_ref[...]).astype(jnp.bfloat16)
            wlo_ref[pl.ds(r, SB), :] = _fp4_bf16(qi & 15) * se
            whi_ref[pl.ds(r, SB), :] = _fp4_bf16(qi >> 4) * se
            return 0
        jax.lax.fori_loop(0, CI // SB, dq_gu, 0)

        def dq_d(i, _):
            r = i * SB
            di = dq_ref[pl.ds(r, SB), :].astype(jnp.int32)
            sf = jnp.exp2(ds_ref[pl.ds(r, SB), :].astype(jnp.float32) - 127.0)
            se = _mmf(sf.astype(jnp.bfloat16), r16b_ref[...]).astype(jnp.bfloat16)
            wdlo_ref[pl.ds(r, SB), :] = _fp4_bf16(di & 15) * se
            wdhi_ref[pl.ds(r, SB), :] = _fp4_bf16(di >> 4) * se
            return 0
        jax.lax.fori_loop(0, H // SB, dq_d, 0)

        x = xs_ref[...].astype(jnp.bfloat16)
        x_lo = x[:, :Hh]
        x_hi = x[:, Hh:]

        gu = jax.lax.dot_general(x_lo, wlo_ref[...], _DN, preferred_element_type=jnp.float32)
        gu += jax.lax.dot_general(x_hi, whi_ref[...], _DN, preferred_element_type=jnp.float32)
        gu += bgu_ref[...]

        gate = jnp.minimum(gu, LIMIT)
        up = jnp.clip(gu, -LIMIT, LIMIT)
        glu = gate * jax.nn.sigmoid(gate * ALPHA)
        up1 = jnp.roll(up, -1, axis=1)
        gated = ((up1 + 1.0) * glu).astype(jnp.bfloat16)

        g_lo = _mmf(gated, clo_ref[...]).astype(jnp.bfloat16)
        g_hi = _mmf(gated, chi_ref[...]).astype(jnp.bfloat16)

        part = jax.lax.dot_general(g_lo, wdlo_ref[...], _DN, preferred_element_type=jnp.float32)
        part += jax.lax.dot_general(g_hi, wdhi_ref[...], _DN, preferred_element_type=jnp.float32)

        @pl.when(c == 0)
        def _():
            ys_ref[...] = part

        @pl.when(c > 0)
        def _():
            ys_ref[...] += part

    @pl.when((nrows > 0) & (c == NC - 1))
    def _():
        ys_ref[...] += bd_ref[...]

        def scatter(i, _):
            tok = tok_ref[base + i]
            row = ys_ref[pl.ds(i, 1), :] * wgt_ref[base + i]
            out_ref[pl.ds(tok, 1), :] += row
            return 0
        jax.lax.fori_loop(0, nrows, scatter, 0)


def kernel(hidden_states, router_indices, routing_weights,
           gate_up_qweight, gate_up_scales, down_qweight, down_scales,
           gate_up_proj_bias, down_proj_bias):
    T, H = hidden_states.shape
    E = gate_up_qweight.shape[0]
    I = down_qweight.shape[2] * 2
    TOPK = router_indices.shape[1]
    P = T * TOPK
    TM = 128
    NT = P // TM + E
    Hh = H // 2
    I2 = 2 * I
    CI = I2 // NC
    CB = CI // 4

    flat = router_indices.reshape(-1).astype(jnp.int32)
    order = jnp.argsort(flat).astype(jnp.int32)
    tok = (order // TOPK).astype(jnp.int32)
    wgt = routing_weights.reshape(-1)[order]
    counts = jnp.zeros((E,), jnp.int32).at[flat].add(1)
    offsets = jnp.cumsum(counts) - counts
    ntiles = (counts + TM - 1) // TM
    tcum = jnp.cumsum(ntiles)
    first_tile = tcum - ntiles
    tr = jnp.arange(NT, dtype=jnp.int32)
    e_of_t = jnp.clip(jnp.searchsorted(tcum, tr, side="right"), 0, E - 1).astype(jnp.int32)
    local = tr - first_tile[e_of_t]
    base = jnp.clip(offsets[e_of_t] + local * TM, 0, P - 1).astype(jnp.int32)
    nrows = jnp.clip(counts[e_of_t] - local * TM, 0, TM).astype(jnp.int32)

    hidden_de = jnp.swapaxes(hidden_states.reshape(T, Hh, 2), 1, 2).reshape(T, H)
    ds_t = down_scales.reshape(E, H, NC, I // (32 * NC)).transpose(0, 2, 1, 3)
    bgu = gate_up_proj_bias.reshape(E, 1, I2)
    bd = down_proj_bias.reshape(E, 1, H)

    r16 = (jnp.arange(Hh)[None, :] // 16 == jnp.arange(Hh // 16)[:, None]).astype(jnp.bfloat16)
    r16b = (jnp.arange(CB)[None, :] // 16 == jnp.arange(CB // 16)[:, None]).astype(jnp.bfloat16)
    clo = (jnp.arange(CI)[:, None] == 4 * jnp.arange(CB)[None, :]).astype(jnp.bfloat16)
    chi = (jnp.arange(CI)[:, None] == 4 * jnp.arange(CB)[None, :] + 2).astype(jnp.bfloat16)

    fixed = lambda t, c, *_: (0, 0)
    grid_spec = pltpu.PrefetchScalarGridSpec(
        num_scalar_prefetch=5,
        grid=(NT, NC),
        in_specs=[
            pl.BlockSpec((T, H), fixed),
            pl.BlockSpec((None, CI, Hh), lambda t, c, e_r, *_: (e_r[t], c, 0)),
            pl.BlockSpec((None, CI, Hh // 16), lambda t, c, e_r, *_: (e_r[t], c, 0)),
            pl.BlockSpec((None, H, CB), lambda t, c, e_r, *_: (e_r[t], 0, c)),
            pl.BlockSpec((None, None, H, CB // 16), lambda t, c, e_r, *_: (e_r[t], c, 0, 0)),
            pl.BlockSpec((None, 1, CI), lambda t, c, e_r, *_: (e_r[t], 0, c)),
            pl.BlockSpec((None, 1, H), lambda t, c, e_r, *_: (e_r[t], 0, 0)),
            pl.BlockSpec((Hh // 16, Hh), fixed),
            pl.BlockSpec((CB // 16, CB), fixed),
            pl.BlockSpec((CI, CB), fixed),
            pl.BlockSpec((CI, CB), fixed),
        ],
        out_specs=pl.BlockSpec((T, H), fixed),
        scratch_shapes=[
            pltpu.VMEM((TM, H), jnp.float32),
            pltpu.VMEM((TM, H), jnp.float32),
            pltpu.VMEM((CI, Hh), jnp.bfloat16),
            pltpu.VMEM((CI, Hh), jnp.bfloat16),
            pltpu.VMEM((H, CB), jnp.bfloat16),
            pltpu.VMEM((H, CB), jnp.bfloat16),
        ],
    )

    body = functools.partial(_moe_body, TM, T, H, I)
    out = pl.pallas_call(
        body,
        grid_spec=grid_spec,
        out_shape=jax.ShapeDtypeStruct((T, H), jnp.float32),
        compiler_params=pltpu.CompilerParams(
            dimension_semantics=("arbitrary", "arbitrary"),
        ),
    )(e_of_t, base, nrows, tok, wgt,
      hidden_de, gate_up_qweight, gate_up_scales, down_qweight, ds_t,
      bgu, bd, r16, r16b, clo, chi)
    return out


# speedup vs baseline: 13.1710x; 1.0969x over previous
import functools

import jax
import jax.numpy as jnp
from jax.experimental import pallas as pl
from jax.experimental.pallas import tpu as pltpu

ALPHA = 1.702
LIMIT = 7.0
NC = 2


def _fp4_bf16(c):
    sign = (c & 8) << 28
    m = c & 7
    e = m >> 1
    m0 = m & 1
    norm = ((126 + e) << 23) | (m0 << 22)
    sub = jnp.where(m0 == 1, jnp.full_like(c, 126 << 23), jnp.zeros_like(c))
    bits = sign | jnp.where(e == 0, sub, norm)
    return jax.lax.bitcast_convert_type(bits, jnp.float32).astype(jnp.bfloat16)


_DN = (((1,), (1,)), ((), ()))
_DS = (((1,), (0,)), ((), ()))


def _mmf(a, b):
    return jax.lax.dot_general(a, b, _DS, preferred_element_type=jnp.float32)


def _moe_body(TM, T, H, I,
              e_ref, base_ref, nrows_ref, tok_ref, wgt_ref,
              hid_ref, q_ref, s_ref, dq_ref, ds_ref, bgu_ref, bd_ref,
              r16_ref, r16b_ref, clo_ref, chi_ref,
              out_ref, xs_ref, ys_ref, wlo_ref, whi_ref, wdlo_ref, wdhi_ref):
    t = pl.program_id(0)
    c = pl.program_id(1)
    Hh = H // 2
    CI = 2 * I // NC
    CB = CI // 4
    SB = 128

    @pl.when((t == 0) & (c == 0))
    def _():
        out_ref[...] = jnp.zeros_like(out_ref)

    nrows = nrows_ref[t]
    base = base_ref[t]

    @pl.when((nrows > 0) & (c == 0))
    def _():
        def gather(i, _):
            xs_ref[pl.ds(i, 1), :] = hid_ref[pl.ds(tok_ref[base + i], 1), :]
            return 0
        jax.lax.fori_loop(0, nrows, gather, 0)

    @pl.when(nrows > 0)
    def _():
        def dq_gu(i, _):
            r = i * SB
            qi = q_ref[pl.ds(r, SB), :].astype(jnp.int32)
            sf = jnp.exp2(s_ref[pl.ds(r, SB), :].astype(jnp.float32) - 127.0)
            se = _mmf(sf.astype(jnp.bfloat16), r16_ref[...]).astype(jnp.bfloat16)
            wlo_ref[pl.ds(r, SB), :] = _fp4_bf16(qi & 15) * se
            whi_ref[pl.ds(r, SB), :] = _fp4_bf16(qi >> 4) * se
            return 0
        jax.lax.fori_loop(0, CI // SB, dq_gu, 0)

        def dq_d(i, _):
            r = i * SB
            di = dq_ref[pl.ds(r, SB), :].astype(jnp.int32)
            sf = jnp.exp2(ds_ref[pl.ds(r, SB), :].astype(jnp.float32) - 127.0)
            se = _mmf(sf.astype(jnp.bfloat16), r16b_ref[...]).astype(jnp.bfloat16)
            wdlo_ref[pl.ds(r, SB), :] = _fp4_bf16(di & 15) * se
            wdhi_ref[pl.ds(r, SB), :] = _fp4_bf16(di >> 4) * se
            return 0
        jax.lax.fori_loop(0, H // SB, dq_d, 0)

        x = xs_ref[...].astype(jnp.bfloat16)
        x_lo = x[:, :Hh]
        x_hi = x[:, Hh:]

        gu = jax.lax.dot_general(x_lo, wlo_ref[...], _DN, preferred_element_type=jnp.float32)
        gu += jax.lax.dot_general(x_hi, whi_ref[...], _DN, preferred_element_type=jnp.float32)
        gu += bgu_ref[...]

        gate = jnp.minimum(gu, LIMIT)
        up = jnp.clip(gu, -LIMIT, LIMIT)
        glu = gate * jax.nn.sigmoid(gate * ALPHA)
        up1 = jnp.roll(up, -1, axis=1)
        gated = ((up1 + 1.0) * glu).astype(jnp.bfloat16)

        g_lo = _mmf(gated, clo_ref[...]).astype(jnp.bfloat16)
        g_hi = _mmf(gated, chi_ref[...]).astype(jnp.bfloat16)

        part = jax.lax.dot_general(g_lo, wdlo_ref[...], _DN, preferred_element_type=jnp.float32)
        part += jax.lax.dot_general(g_hi, wdhi_ref[...], _DN, preferred_element_type=jnp.float32)

        @pl.when(c == 0)
        def _():
            ys_ref[...] = part

        @pl.when(c > 0)
        def _():
            ys_ref[...] += part

    @pl.when((nrows > 0) & (c == NC - 1))
    def _():
        ys_ref[...] += bd_ref[...]

        def scatter(i, _):
            tok = tok_ref[base + i]
            row = ys_ref[pl.ds(i, 1), :] * wgt_ref[base + i]
            out_ref[pl.ds(tok, 1), :] += row
            return 0
        jax.lax.fori_loop(0, nrows, scatter, 0)


def kernel(hidden_states, router_indices, routing_weights,
           gate_up_qweight, gate_up_scales, down_qweight, down_scales,
           gate_up_proj_bias, down_proj_bias):
    T, H = hidden_states.shape
    E = gate_up_qweight.shape[0]
    I = down_qweight.shape[2] * 2
    TOPK = router_indices.shape[1]
    P = T * TOPK
    TM = 128
    NT = P // TM + E
    Hh = H // 2
    I2 = 2 * I
    CI = I2 // NC
    CB = CI // 4

    flat = router_indices.reshape(-1).astype(jnp.int32)
    order = jnp.argsort(flat).astype(jnp.int32)
    tok = (order // TOPK).astype(jnp.int32)
    wgt = routing_weights.reshape(-1)[order]
    counts = jnp.zeros((E,), jnp.int32).at[flat].add(1)
    offsets = jnp.cumsum(counts) - counts
    ntiles = (counts + TM - 1) // TM
    tcum = jnp.cumsum(ntiles)
    first_tile = tcum - ntiles
    tr = jnp.arange(NT, dtype=jnp.int32)
    e_of_t = jnp.clip(jnp.searchsorted(tcum, tr, side="right"), 0, E - 1).astype(jnp.int32)
    local = tr - first_tile[e_of_t]
    base = jnp.clip(offsets[e_of_t] + local * TM, 0, P - 1).astype(jnp.int32)
    nrows = jnp.clip(counts[e_of_t] - local * TM, 0, TM).astype(jnp.int32)

    hidden_de = jnp.swapaxes(hidden_states.reshape(T, Hh, 2), 1, 2).reshape(T, H)
    ds_t = down_scales.reshape(E, H, NC, I // (32 * NC)).transpose(0, 2, 1, 3)
    bgu = gate_up_proj_bias.reshape(E, 1, I2)
    bd = down_proj_bias.reshape(E, 1, H)

    r16 = (jnp.arange(Hh)[None, :] // 16 == jnp.arange(Hh // 16)[:, None]).astype(jnp.bfloat16)
    r16b = (jnp.arange(CB)[None, :] // 16 == jnp.arange(CB // 16)[:, None]).astype(jnp.bfloat16)
    clo = (jnp.arange(CI)[:, None] == 4 * jnp.arange(CB)[None, :]).astype(jnp.bfloat16)
    chi = (jnp.arange(CI)[:, None] == 4 * jnp.arange(CB)[None, :] + 2).astype(jnp.bfloat16)

    fixed = lambda t, c, *_: (0, 0)
    grid_spec = pltpu.PrefetchScalarGridSpec(
        num_scalar_prefetch=5,
        grid=(NT, NC),
        in_specs=[
            pl.BlockSpec((T, H), fixed),
            pl.BlockSpec((None, CI, Hh), lambda t, c, e_r, *_: (e_r[t], c, 0)),
            pl.BlockSpec((None, CI, Hh // 16), lambda t, c, e_r, *_: (e_r[t], c, 0)),
            pl.BlockSpec((None, H, CB), lambda t, c, e_r, *_: (e_r[t], 0, c)),
            pl.BlockSpec((None, None, H, CB // 16), lambda t, c, e_r, *_: (e_r[t], c, 0, 0)),
            pl.BlockSpec((None, 1, CI), lambda t, c, e_r, *_: (e_r[t], 0, c)),
            pl.BlockSpec((None, 1, H), lambda t, c, e_r, *_: (e_r[t], 0, 0)),
            pl.BlockSpec((Hh // 16, Hh), fixed),
            pl.BlockSpec((CB // 16, CB), fixed),
            pl.BlockSpec((CI, CB), fixed),
            pl.BlockSpec((CI, CB), fixed),
        ],
        out_specs=pl.BlockSpec((T, H), fixed),
        scratch_shapes=[
            pltpu.VMEM((TM, H), jnp.float32),
            pltpu.VMEM((TM, H), jnp.float32),
            pltpu.VMEM((CI, Hh), jnp.bfloat16),
            pltpu.VMEM((CI, Hh), jnp.bfloat16),
            pltpu.VMEM((H, CB), jnp.bfloat16),
            pltpu.VMEM((H, CB), jnp.bfloat16),
        ],
    )

    body = functools.partial(_moe_body, TM, T, H, I)
    out = pl.pallas_call(
        body,
        grid_spec=grid_spec,
        out_shape=jax.ShapeDtypeStruct((T, H), jnp.float32),
        compiler_params=pltpu.CompilerParams(
            dimension_semantics=("arbitrary", "arbitrary"),
        ),
    )(e_of_t, base, nrows, tok, wgt,
      hidden_de, gate_up_qweight, gate_up_scales, down_qweight, ds_t,
      bgu, bd, r16, r16b, clo, chi)
    return out
